# baseline (speedup 1.0000x reference)
import functools

import jax
import jax.numpy as jnp
from jax import lax
from jax.experimental import pallas as pl
from jax.experimental.pallas import tpu as pltpu
from jax.experimental.pallas import tpu_sc as plsc

N = 10000
NP = 10240
E = 640000
HID = 128
ACT = 32
FEAT = 256
B = 64

NC = 2
NS = 16
NW = NC * NS
CH = 128
NCHK = 158
EPW = NCHK * CH
EPAD = NW * EPW
PAD_NODE = N
NODES_PER_TILE = NP // NS

_mesh = plsc.VectorSubcoreMesh(core_axis_name="c", subcore_axis_name="s",
                              num_cores=NC, num_subcores=NS)
_sc_params = pltpu.CompilerParams(needs_layout_passes=False,
                                  use_tc_tiling_on_sc=False)


def _leaky(v):
    return jnp.where(v > 0, v, 0.2 * v)


_GATHER_DNUMS = lax.GatherDimensionNumbers(
    offset_dims=(), collapsed_slice_dims=(0,), start_index_map=(0,))


def _splat_lane(vec, j):
    idx = jnp.full((16, 1), j, jnp.int32)
    return lax.gather(vec, idx, _GATHER_DNUMS, (1,),
                      mode=lax.GatherScatterMode.PROMISE_IN_BOUNDS)


def _allsum16(vec):
    lanes = lax.iota(jnp.int32, 16)
    for sh in (8, 4, 2, 1):
        idx = ((lanes + sh) & 15)[:, None]
        vec = vec + lax.gather(vec, idx, _GATHER_DNUMS, (1,),
                               mode=lax.GatherScatterMode.PROMISE_IN_BOUNDS)
    return vec


def _sc_pass_a(src_hbm, dst_hbm, x_hbm, w1_hbm, a1s_hbm, a1d_hbm,
               den_out, snum_out,
               src_v, dst_v, x_v, par_v, acc_d, acc_s):
    cid = lax.axis_index("c")
    sid = lax.axis_index("s")
    wid = cid * NS + sid

    pltpu.sync_copy(src_hbm.at[wid], src_v)
    pltpu.sync_copy(dst_hbm.at[wid], dst_v)
    pltpu.sync_copy(x_hbm, x_v)
    pltpu.sync_copy(w1_hbm, par_v.at[0])
    pltpu.sync_copy(a1s_hbm, par_v.at[1])
    pltpu.sync_copy(a1d_hbm, par_v.at[2])

    def _dot(row):
        def body(i, t):
            return t + par_v[0, pl.ds(i * 16, 16)] * par_v[row, pl.ds(i * 16, 16)]
        return _allsum16(lax.fori_loop(0, HID // 16, body,
                                       jnp.zeros((16,), jnp.float32)))
    c1s = _dot(1)
    c1d = _dot(2)

    def zbody(i, _):
        z = jnp.zeros((16,), jnp.float32)
        acc_d[pl.ds(i * 16, 16)] = z
        acc_s[pl.ds(i * 16, 16)] = z
        return 0
    lax.fori_loop(0, NP // 16, zbody, 0)

    def ebody(k, _):
        ch = k // (CH // 16)
        g = k % (CH // 16)
        s16 = src_v[ch, pl.ds(g * 16, 16)]
        d16 = dst_v[ch, pl.ds(g * 16, 16)]
        xs = plsc.load_gather(x_v, [s16])
        xd = plsc.load_gather(x_v, [d16])
        w = jnp.exp(_leaky(c1s * xs + c1d * xd))
        plsc.addupdate_scatter(acc_d, [d16], w)
        plsc.addupdate_scatter(acc_s, [d16], w * xs)
        return 0
    lax.fori_loop(0, NCHK * (CH // 16), ebody, 0)

    pltpu.sync_copy(acc_d, den_out.at[wid])
    pltpu.sync_copy(acc_s, snum_out.at[wid])


def _run_pass_a(src3d, dst3d, xp, w1row, a1s, a1d):
    kfn = pl.kernel(
        _sc_pass_a,
        out_type=(
            jax.ShapeDtypeStruct((NW, NP), jnp.float32),
            jax.ShapeDtypeStruct((NW, NP), jnp.float32),
        ),
        mesh=_mesh,
        compiler_params=_sc_params,
        scratch_types=[
            pltpu.VMEM((NCHK, CH), jnp.int32),
            pltpu.VMEM((NCHK, CH), jnp.int32),
            pltpu.VMEM((NP,), jnp.float32),
            pltpu.VMEM((3, HID), jnp.float32),
            pltpu.VMEM((NP,), jnp.float32),
            pltpu.VMEM((NP,), jnp.float32),
        ],
    )
    return kfn(src3d, dst3d, xp, w1row, a1s, a1d)


BLK = 1024


def _tc_pass_b(x_ref, denp_ref, snump_ref, w1t_ref, a1st_ref, a1dt_ref,
               b1t_ref, w2t_ref, a2st_ref, a2dt_ref,
               gt_ref, es2_ref, ed2_ref, den2i_ref, num2it_ref):
    xb = x_ref[...]
    w1t = w1t_ref[...]
    c1s = jnp.sum(w1t * a1st_ref[...])
    c1d = jnp.sum(w1t * a1dt_ref[...])
    den_e = jnp.sum(denp_ref[...], axis=0, keepdims=True)
    snum_e = jnp.sum(snump_ref[...], axis=0, keepdims=True)
    wself = jnp.exp(_leaky((c1s + c1d) * xb))
    den = den_e + wself
    snum = snum_e + wself * xb
    s1 = snum / (den + 1e-16)
    h1t = jax.nn.relu(w1t * s1 + b1t_ref[...])
    gt = jnp.dot(w2t_ref[...], h1t, preferred_element_type=jnp.float32)
    es2 = jnp.sum(gt * a2st_ref[...], axis=0, keepdims=True)
    ed2 = jnp.sum(gt * a2dt_ref[...], axis=0, keepdims=True)
    w2self = jnp.exp(_leaky(es2 + ed2))
    gt_ref[...] = gt
    es2_ref[...] = es2
    ed2_ref[...] = ed2
    den2i_ref[...] = w2self
    num2it_ref[...] = w2self * gt


def _run_pass_b(xrow, denp, snump, w1t, a1st, a1dt, b1t, W2t, a2st, a2dt):
    nblk = NP // BLK
    full = lambda shape: pl.BlockSpec(shape, lambda i: (0,) * len(shape))
    return pl.pallas_call(
        _tc_pass_b,
        grid=(nblk,),
        in_specs=[
            pl.BlockSpec((1, BLK), lambda i: (0, i)),
            pl.BlockSpec((NW, BLK), lambda i: (0, i)),
            pl.BlockSpec((NW, BLK), lambda i: (0, i)),
            full((HID, 1)), full((HID, 1)), full((HID, 1)),
            full((HID, 1)), full((ACT, HID)), full((ACT, 1)), full((ACT, 1)),
        ],
        out_specs=[
            pl.BlockSpec((ACT, BLK), lambda i: (0, i)),
            pl.BlockSpec((1, BLK), lambda i: (0, i)),
            pl.BlockSpec((1, BLK), lambda i: (0, i)),
            pl.BlockSpec((1, BLK), lambda i: (0, i)),
            pl.BlockSpec((ACT, BLK), lambda i: (0, i)),
        ],
        out_shape=(
            jax.ShapeDtypeStruct((ACT, NP), jnp.float32),
            jax.ShapeDtypeStruct((1, NP), jnp.float32),
            jax.ShapeDtypeStruct((1, NP), jnp.float32),
            jax.ShapeDtypeStruct((1, NP), jnp.float32),
            jax.ShapeDtypeStruct((ACT, NP), jnp.float32),
        ),
    )(xrow, denp, snump, w1t, a1st, a1dt, b1t, W2t, a2st, a2dt)


def _sc_pass_c(src_hbm, dst_hbm, es2_hbm, ed2_hbm, g_hbm,
               den_out, num_out,
               src_v, dst_v, es_v, ed_v, acc_d,
               rows_in0, rows_in1, rows_out0, rows_out1, slab_v, num_sh,
               sg0, sg1, ss0, ss1):
    cid = lax.axis_index("c")
    sid = lax.axis_index("s")
    wid = cid * NS + sid

    pltpu.sync_copy(src_hbm.at[wid], src_v)
    pltpu.sync_copy(dst_hbm.at[wid], dst_v)
    pltpu.sync_copy(es2_hbm, es_v)
    pltpu.sync_copy(ed2_hbm, ed_v)

    def zbody(i, _):
        acc_d[pl.ds(i * 16, 16)] = jnp.zeros((16,), jnp.float32)
        return 0
    lax.fori_loop(0, NP // 16, zbody, 0)

    def zslab(i, _):
        slab_v[i // 2, pl.ds((i % 2) * 16, 16)] = jnp.zeros((16,), jnp.float32)
        return 0
    lax.fori_loop(0, NODES_PER_TILE * 2, zslab, 0)
    pltpu.sync_copy(slab_v, num_sh.at[pl.ds(sid * NODES_PER_TILE,
                                            NODES_PER_TILE)])
    plsc.subcore_barrier()

    rows_in = (rows_in0, rows_in1)
    rows_out = (rows_out0, rows_out1)
    sg = (sg0, sg1)
    ss = (ss0, ss1)

    for b in range(2):
        pltpu.async_copy(g_hbm.at[src_v.at[b]], rows_in[b], sg[b])

    def chunk_pair(it, _):
        for b in range(2):
            ch = it * 2 + b
            pltpu.make_async_copy(g_hbm.at[src_v.at[ch]], rows_in[b],
                                  sg[b]).wait()

            def gbody(gg, _):
                s16 = src_v[ch, pl.ds(gg * 16, 16)]
                d16 = dst_v[ch, pl.ds(gg * 16, 16)]
                es = plsc.load_gather(es_v, [s16])
                ed = plsc.load_gather(ed_v, [d16])
                w16 = jnp.exp(_leaky(es + ed))
                plsc.addupdate_scatter(acc_d, [d16], w16)
                for j in range(16):
                    e = gg * 16 + j
                    wsp = _splat_lane(w16, j)
                    rows_out[b][e, pl.ds(0, 16)] = (
                        rows_in[b][e, pl.ds(0, 16)] * wsp)
                    rows_out[b][e, pl.ds(16, 16)] = (
                        rows_in[b][e, pl.ds(16, 16)] * wsp)
                return 0
            lax.fori_loop(0, CH // 16, gbody, 0)

            @pl.when(ch >= 2)
            def _():
                pltpu.make_async_copy(rows_out[b], num_sh.at[dst_v.at[ch]],
                                      ss[b]).wait()
            pltpu.async_copy(rows_out[b], num_sh.at[dst_v.at[ch]], ss[b],
                             add=True)

            @pl.when(ch + 2 < NCHK)
            def _():
                pltpu.async_copy(g_hbm.at[src_v.at[ch + 2]], rows_in[b],
                                 sg[b])
        return 0
    lax.fori_loop(0, NCHK // 2, chunk_pair, 0)

    for b in range(2):
        pltpu.make_async_copy(rows_out[b], num_sh.at[dst_v.at[NCHK - 2 + b]],
                              ss[b]).wait()

    pltpu.sync_copy(acc_d, den_out.at[wid])
    plsc.subcore_barrier()

    pltpu.sync_copy(num_sh.at[pl.ds(sid * NODES_PER_TILE, NODES_PER_TILE)],
                    slab_v)
    pltpu.sync_copy(slab_v,
                    num_out.at[cid, pl.ds(sid * NODES_PER_TILE,
                                          NODES_PER_TILE)])


def _run_pass_c(src3d, dst3d, es2f, ed2f, g):
    kfn = pl.kernel(
        _sc_pass_c,
        out_type=(
            jax.ShapeDtypeStruct((NW, NP), jnp.float32),
            jax.ShapeDtypeStruct((NC, NP, ACT), jnp.float32),
        ),
        mesh=_mesh,
        compiler_params=_sc_params,
        scratch_types=[
            pltpu.VMEM((NCHK, CH), jnp.int32),
            pltpu.VMEM((NCHK, CH), jnp.int32),
            pltpu.VMEM((NP,), jnp.float32),
            pltpu.VMEM((NP,), jnp.float32),
            pltpu.VMEM((NP,), jnp.float32),
            pltpu.VMEM((CH, ACT), jnp.float32),
            pltpu.VMEM((CH, ACT), jnp.float32),
            pltpu.VMEM((CH, ACT), jnp.float32),
            pltpu.VMEM((CH, ACT), jnp.float32),
            pltpu.VMEM((NODES_PER_TILE, ACT), jnp.float32),
            pltpu.VMEM_SHARED((NP, ACT), jnp.float32),
            pltpu.SemaphoreType.DMA,
            pltpu.SemaphoreType.DMA,
            pltpu.SemaphoreType.DMA,
            pltpu.SemaphoreType.DMA,
        ],
    )
    return kfn(src3d, dst3d, es2f, ed2f, g)


def _tc_pass_d(numps_ref, denp_ref, den2i_ref, num2i_ref, b2t_ref,
               gammat_ref, betat_ref, feat_ref, wv1_ref, bv1_ref,
               wv2_ref, bv2_ref, actor_ref, critic_ref):
    i = pl.program_id(0)
    num = jnp.sum(numps_ref[...], axis=0) + num2i_ref[...]
    den = (jnp.sum(denp_ref[...], axis=0, keepdims=True)
           + den2i_ref[...])
    out2 = num / (den + 1e-16) + b2t_ref[...]
    h2 = jax.nn.relu(out2)
    mu = jnp.mean(h2, axis=0, keepdims=True)
    var = jnp.mean((h2 - mu) ** 2, axis=0, keepdims=True)
    ln = (h2 - mu) / jnp.sqrt(var + 1e-5) * gammat_ref[...] + betat_ref[...]
    cols = i * BLK + lax.broadcasted_iota(jnp.int32, (1, BLK), 1)
    ln = jnp.where(cols < N, ln, 0.0)
    part = jnp.sum(ln, axis=-1, keepdims=True)

    @pl.when(i == 0)
    def _():
        actor_ref[...] = part
        v = jax.nn.relu(jnp.dot(feat_ref[...], wv1_ref[...],
                                preferred_element_type=jnp.float32)
                        + bv1_ref[...])
        critic_ref[...] = jax.nn.relu(
            jnp.dot(v, wv2_ref[...], preferred_element_type=jnp.float32)
            + bv2_ref[...])

    @pl.when(i > 0)
    def _():
        actor_ref[...] = actor_ref[...] + part


def _run_pass_d(num2pst, den2p, den2i, num2it, b2t, gammat, betat,
                features, Wv1, bv12, Wv2, bv22):
    nblk = NP // BLK
    full = lambda shape: pl.BlockSpec(shape, lambda i: (0,) * len(shape))
    return pl.pallas_call(
        _tc_pass_d,
        grid=(nblk,),
        in_specs=[
            pl.BlockSpec((NC, ACT, BLK), lambda i: (0, 0, i)),
            pl.BlockSpec((NW, BLK), lambda i: (0, i)),
            pl.BlockSpec((1, BLK), lambda i: (0, i)),
            pl.BlockSpec((ACT, BLK), lambda i: (0, i)),
            full((ACT, 1)), full((ACT, 1)), full((ACT, 1)),
            full((B, FEAT)), full((FEAT, 64)), full((1, 64)),
            full((64, 64)), full((1, 64)),
        ],
        out_specs=[
            pl.BlockSpec((ACT, 1), lambda i: (0, 0)),
            pl.BlockSpec((B, 64), lambda i: (0, 0)),
        ],
        out_shape=(
            jax.ShapeDtypeStruct((ACT, 1), jnp.float32),
            jax.ShapeDtypeStruct((B, 64), jnp.float32),
        ),
    )(num2pst, den2p, den2i, num2it, b2t, gammat, betat,
      features, Wv1, bv12, Wv2, bv22)


@jax.jit
def kernel(features, x, edge_index, W1, a1s, a1d, b1, W2, a2s, a2d, b2,
           gamma, beta, Wv1, bv1, Wv2, bv2):
    pad_e = jnp.full((EPAD - E,), PAD_NODE, jnp.int32)
    src3d = jnp.concatenate([edge_index[0], pad_e]).reshape(NW, NCHK, CH)
    dst3d = jnp.concatenate([edge_index[1], pad_e]).reshape(NW, NCHK, CH)
    xp = jnp.concatenate([x[:, 0], jnp.zeros((NP - N,), jnp.float32)])

    denp, snump = _run_pass_a(src3d, dst3d, xp, W1.reshape(HID), a1s, a1d)

    gt, es2, ed2, den2i, num2it = _run_pass_b(
        xp.reshape(1, NP), denp, snump,
        W1.reshape(HID, 1), a1s.reshape(HID, 1), a1d.reshape(HID, 1),
        b1.reshape(HID, 1), W2.T, a2s.reshape(ACT, 1), a2d.reshape(ACT, 1))

    den2p, num2ps = _run_pass_c(src3d, dst3d,
                                es2.reshape(NP), ed2.reshape(NP), gt.T)

    actorT, critic = _run_pass_d(
        jnp.transpose(num2ps, (0, 2, 1)), den2p, den2i, num2it,
        b2.reshape(ACT, 1), gamma.reshape(ACT, 1), beta.reshape(ACT, 1),
        features, Wv1, bv1.reshape(1, 64), Wv2, bv2.reshape(1, 64))
    return (actorT.T, critic)


# speedup vs baseline: 1.0338x; 1.0338x over previous
import functools

import jax
import jax.numpy as jnp
from jax import lax
from jax.experimental import pallas as pl
from jax.experimental.pallas import tpu as pltpu
from jax.experimental.pallas import tpu_sc as plsc

N = 10000
NP = 10240
E = 640000
HID = 128
ACT = 32
FEAT = 256
B = 64

NC = 2
NS = 16
NW = NC * NS
CH = 128
NCHK = 160
EPW = NCHK * CH
EPAD = NW * EPW
PAD_NODE = N
NODES_PER_TILE = NP // NS

_mesh = plsc.VectorSubcoreMesh(core_axis_name="c", subcore_axis_name="s",
                              num_cores=NC, num_subcores=NS)
_sc_params = pltpu.CompilerParams(needs_layout_passes=False,
                                  use_tc_tiling_on_sc=False)


def _leaky(v):
    return jnp.where(v > 0, v, 0.2 * v)


_GATHER_DNUMS = lax.GatherDimensionNumbers(
    offset_dims=(), collapsed_slice_dims=(0,), start_index_map=(0,))


def _splat_lane(vec, j):
    idx = jnp.full((16, 1), j, jnp.int32)
    return lax.gather(vec, idx, _GATHER_DNUMS, (1,),
                      mode=lax.GatherScatterMode.PROMISE_IN_BOUNDS)


def _allsum16(vec):
    lanes = lax.iota(jnp.int32, 16)
    for sh in (8, 4, 2, 1):
        idx = ((lanes + sh) & 15)[:, None]
        vec = vec + lax.gather(vec, idx, _GATHER_DNUMS, (1,),
                               mode=lax.GatherScatterMode.PROMISE_IN_BOUNDS)
    return vec


def _sc_pass_a(src_hbm, dst_hbm, x_hbm, w1_hbm, a1s_hbm, a1d_hbm,
               den_out, snum_out,
               src_v, dst_v, x_v, par_v, acc_d, acc_s):
    cid = lax.axis_index("c")
    sid = lax.axis_index("s")
    wid = cid * NS + sid

    pltpu.sync_copy(src_hbm.at[wid], src_v)
    pltpu.sync_copy(dst_hbm.at[wid], dst_v)
    pltpu.sync_copy(x_hbm, x_v)
    pltpu.sync_copy(w1_hbm, par_v.at[0])
    pltpu.sync_copy(a1s_hbm, par_v.at[1])
    pltpu.sync_copy(a1d_hbm, par_v.at[2])

    def _dot(row):
        def body(i, t):
            return t + par_v[0, pl.ds(i * 16, 16)] * par_v[row, pl.ds(i * 16, 16)]
        return _allsum16(lax.fori_loop(0, HID // 16, body,
                                       jnp.zeros((16,), jnp.float32)))
    c1s = _dot(1)
    c1d = _dot(2)

    def zbody(i, _):
        z = jnp.zeros((16,), jnp.float32)
        acc_d[pl.ds(i * 16, 16)] = z
        acc_s[pl.ds(i * 16, 16)] = z
        return 0
    lax.fori_loop(0, NP // 16, zbody, 0)

    def ebody(k, _):
        ch = k // (CH // 16)
        g = k % (CH // 16)
        s16 = src_v[ch, pl.ds(g * 16, 16)]
        d16 = dst_v[ch, pl.ds(g * 16, 16)]
        xs = plsc.load_gather(x_v, [s16])
        xd = plsc.load_gather(x_v, [d16])
        w = jnp.exp(_leaky(c1s * xs + c1d * xd))
        plsc.addupdate_scatter(acc_d, [d16], w)
        plsc.addupdate_scatter(acc_s, [d16], w * xs)
        return 0
    lax.fori_loop(0, NCHK * (CH // 16), ebody, 0)

    pltpu.sync_copy(acc_d, den_out.at[wid])
    pltpu.sync_copy(acc_s, snum_out.at[wid])


def _run_pass_a(src3d, dst3d, xp, w1row, a1s, a1d):
    kfn = pl.kernel(
        _sc_pass_a,
        out_type=(
            jax.ShapeDtypeStruct((NW, NP), jnp.float32),
            jax.ShapeDtypeStruct((NW, NP), jnp.float32),
        ),
        mesh=_mesh,
        compiler_params=_sc_params,
        scratch_types=[
            pltpu.VMEM((NCHK, CH), jnp.int32),
            pltpu.VMEM((NCHK, CH), jnp.int32),
            pltpu.VMEM((NP,), jnp.float32),
            pltpu.VMEM((3, HID), jnp.float32),
            pltpu.VMEM((NP,), jnp.float32),
            pltpu.VMEM((NP,), jnp.float32),
        ],
    )
    return kfn(src3d, dst3d, xp, w1row, a1s, a1d)


BLK = 1024


def _tc_pass_b(x_ref, denp_ref, snump_ref, w1t_ref, a1st_ref, a1dt_ref,
               b1t_ref, w2t_ref, a2st_ref, a2dt_ref,
               gt_ref, es2_ref, ed2_ref, den2i_ref, num2it_ref):
    xb = x_ref[...]
    w1t = w1t_ref[...]
    c1s = jnp.sum(w1t * a1st_ref[...])
    c1d = jnp.sum(w1t * a1dt_ref[...])
    den_e = jnp.sum(denp_ref[...], axis=0, keepdims=True)
    snum_e = jnp.sum(snump_ref[...], axis=0, keepdims=True)
    wself = jnp.exp(_leaky((c1s + c1d) * xb))
    den = den_e + wself
    snum = snum_e + wself * xb
    s1 = snum / (den + 1e-16)
    h1t = jax.nn.relu(w1t * s1 + b1t_ref[...])
    gt = jnp.dot(w2t_ref[...], h1t, preferred_element_type=jnp.float32)
    es2 = jnp.sum(gt * a2st_ref[...], axis=0, keepdims=True)
    ed2 = jnp.sum(gt * a2dt_ref[...], axis=0, keepdims=True)
    w2self = jnp.exp(_leaky(es2 + ed2))
    gt_ref[...] = gt
    es2_ref[...] = es2
    ed2_ref[...] = ed2
    den2i_ref[...] = w2self
    num2it_ref[...] = w2self * gt


def _run_pass_b(xrow, denp, snump, w1t, a1st, a1dt, b1t, W2t, a2st, a2dt):
    nblk = NP // BLK
    full = lambda shape: pl.BlockSpec(shape, lambda i: (0,) * len(shape))
    return pl.pallas_call(
        _tc_pass_b,
        grid=(nblk,),
        in_specs=[
            pl.BlockSpec((1, BLK), lambda i: (0, i)),
            pl.BlockSpec((NW, BLK), lambda i: (0, i)),
            pl.BlockSpec((NW, BLK), lambda i: (0, i)),
            full((HID, 1)), full((HID, 1)), full((HID, 1)),
            full((HID, 1)), full((ACT, HID)), full((ACT, 1)), full((ACT, 1)),
        ],
        out_specs=[
            pl.BlockSpec((ACT, BLK), lambda i: (0, i)),
            pl.BlockSpec((1, BLK), lambda i: (0, i)),
            pl.BlockSpec((1, BLK), lambda i: (0, i)),
            pl.BlockSpec((1, BLK), lambda i: (0, i)),
            pl.BlockSpec((ACT, BLK), lambda i: (0, i)),
        ],
        out_shape=(
            jax.ShapeDtypeStruct((ACT, NP), jnp.float32),
            jax.ShapeDtypeStruct((1, NP), jnp.float32),
            jax.ShapeDtypeStruct((1, NP), jnp.float32),
            jax.ShapeDtypeStruct((1, NP), jnp.float32),
            jax.ShapeDtypeStruct((ACT, NP), jnp.float32),
        ),
    )(xrow, denp, snump, w1t, a1st, a1dt, b1t, W2t, a2st, a2dt)


def _sc_pass_c(src_hbm, dst_hbm, es2_hbm, ed2_hbm, g_hbm,
               den_out, num_out,
               src_v, dst_v, es_v, ed_v, acc_d,
               rows_in0, rows_in1,
               rows_out0, rows_out1, rows_out2, rows_out3, num_sh,
               sg0, sg1, ss0, ss1, ss2, ss3):
    cid = lax.axis_index("c")
    sid = lax.axis_index("s")
    wid = cid * NS + sid

    pltpu.sync_copy(src_hbm.at[wid], src_v)
    pltpu.sync_copy(dst_hbm.at[wid], dst_v)
    pltpu.sync_copy(es2_hbm, es_v)
    pltpu.sync_copy(ed2_hbm, ed_v)

    def zbody(i, _):
        acc_d[pl.ds(i * 16, 16)] = jnp.zeros((16,), jnp.float32)
        return 0
    lax.fori_loop(0, NP // 16, zbody, 0)

    def zrows(i, _):
        rows_out0[i // 2, pl.ds((i % 2) * 16, 16)] = jnp.zeros((16,),
                                                               jnp.float32)
        return 0
    lax.fori_loop(0, CH * 2, zrows, 0)
    for k in range(NODES_PER_TILE // CH):
        pltpu.sync_copy(rows_out0,
                        num_sh.at[pl.ds(sid * NODES_PER_TILE + k * CH, CH)])
    plsc.subcore_barrier()

    rows_in = (rows_in0, rows_in1)
    rows_out = (rows_out0, rows_out1, rows_out2, rows_out3)
    sg = (sg0, sg1)
    ss = (ss0, ss1, ss2, ss3)

    for b in range(2):
        pltpu.async_copy(g_hbm.at[src_v.at[b]], rows_in[b], sg[b])

    def chunk_quad(it, _):
        for q in range(4):
            ch = it * 4 + q
            b = q % 2
            pltpu.make_async_copy(g_hbm.at[src_v.at[ch]], rows_in[b],
                                  sg[b]).wait()

            @pl.when(ch >= 4)
            def _():
                pltpu.make_async_copy(rows_out[q], num_sh.at[dst_v.at[ch]],
                                      ss[q]).wait()

            for gg in range(CH // 16):
                s16 = src_v[ch, pl.ds(gg * 16, 16)]
                d16 = dst_v[ch, pl.ds(gg * 16, 16)]
                es = plsc.load_gather(es_v, [s16])
                ed = plsc.load_gather(ed_v, [d16])
                w16 = jnp.exp(_leaky(es + ed))
                plsc.addupdate_scatter(acc_d, [d16], w16)
                for j in range(16):
                    e = gg * 16 + j
                    wsp = _splat_lane(w16, j)
                    rows_out[q][e, pl.ds(0, 16)] = (
                        rows_in[b][e, pl.ds(0, 16)] * wsp)
                    rows_out[q][e, pl.ds(16, 16)] = (
                        rows_in[b][e, pl.ds(16, 16)] * wsp)

            pltpu.async_copy(rows_out[q], num_sh.at[dst_v.at[ch]], ss[q],
                             add=True)

            @pl.when(ch + 2 < NCHK)
            def _():
                pltpu.async_copy(g_hbm.at[src_v.at[ch + 2]], rows_in[b],
                                 sg[b])
        return 0
    lax.fori_loop(0, NCHK // 4, chunk_quad, 0)

    for q in range(4):
        pltpu.make_async_copy(rows_out[q], num_sh.at[dst_v.at[NCHK - 4 + q]],
                              ss[q]).wait()

    pltpu.sync_copy(acc_d, den_out.at[wid])
    plsc.subcore_barrier()

    for k in range(NODES_PER_TILE // CH):
        pltpu.sync_copy(
            num_sh.at[pl.ds(sid * NODES_PER_TILE + k * CH, CH)], rows_out0)
        pltpu.sync_copy(
            rows_out0,
            num_out.at[cid, pl.ds(sid * NODES_PER_TILE + k * CH, CH)])


def _run_pass_c(src3d, dst3d, es2f, ed2f, g):
    kfn = pl.kernel(
        _sc_pass_c,
        out_type=(
            jax.ShapeDtypeStruct((NW, NP), jnp.float32),
            jax.ShapeDtypeStruct((NC, NP, ACT), jnp.float32),
        ),
        mesh=_mesh,
        compiler_params=_sc_params,
        scratch_types=[
            pltpu.VMEM((NCHK, CH), jnp.int32),
            pltpu.VMEM((NCHK, CH), jnp.int32),
            pltpu.VMEM((NP,), jnp.float32),
            pltpu.VMEM((NP,), jnp.float32),
            pltpu.VMEM((NP,), jnp.float32),
            pltpu.VMEM((CH, ACT), jnp.float32),
            pltpu.VMEM((CH, ACT), jnp.float32),
            pltpu.VMEM((CH, ACT), jnp.float32),
            pltpu.VMEM((CH, ACT), jnp.float32),
            pltpu.VMEM((CH, ACT), jnp.float32),
            pltpu.VMEM((CH, ACT), jnp.float32),
            pltpu.VMEM_SHARED((NP, ACT), jnp.float32),
            pltpu.SemaphoreType.DMA,
            pltpu.SemaphoreType.DMA,
            pltpu.SemaphoreType.DMA,
            pltpu.SemaphoreType.DMA,
            pltpu.SemaphoreType.DMA,
            pltpu.SemaphoreType.DMA,
        ],
    )
    return kfn(src3d, dst3d, es2f, ed2f, g)


def _tc_pass_d(numps_ref, denp_ref, den2i_ref, num2i_ref, b2t_ref,
               gammat_ref, betat_ref, feat_ref, wv1_ref, bv1_ref,
               wv2_ref, bv2_ref, actor_ref, critic_ref):
    i = pl.program_id(0)
    num = jnp.sum(numps_ref[...], axis=0) + num2i_ref[...]
    den = (jnp.sum(denp_ref[...], axis=0, keepdims=True)
           + den2i_ref[...])
    out2 = num / (den + 1e-16) + b2t_ref[...]
    h2 = jax.nn.relu(out2)
    mu = jnp.mean(h2, axis=0, keepdims=True)
    var = jnp.mean((h2 - mu) ** 2, axis=0, keepdims=True)
    ln = (h2 - mu) / jnp.sqrt(var + 1e-5) * gammat_ref[...] + betat_ref[...]
    cols = i * BLK + lax.broadcasted_iota(jnp.int32, (1, BLK), 1)
    ln = jnp.where(cols < N, ln, 0.0)
    part = jnp.sum(ln, axis=-1, keepdims=True)

    @pl.when(i == 0)
    def _():
        actor_ref[...] = part
        v = jax.nn.relu(jnp.dot(feat_ref[...], wv1_ref[...],
                                preferred_element_type=jnp.float32)
                        + bv1_ref[...])
        critic_ref[...] = jax.nn.relu(
            jnp.dot(v, wv2_ref[...], preferred_element_type=jnp.float32)
            + bv2_ref[...])

    @pl.when(i > 0)
    def _():
        actor_ref[...] = actor_ref[...] + part


def _run_pass_d(num2pst, den2p, den2i, num2it, b2t, gammat, betat,
                features, Wv1, bv12, Wv2, bv22):
    nblk = NP // BLK
    full = lambda shape: pl.BlockSpec(shape, lambda i: (0,) * len(shape))
    return pl.pallas_call(
        _tc_pass_d,
        grid=(nblk,),
        in_specs=[
            pl.BlockSpec((NC, ACT, BLK), lambda i: (0, 0, i)),
            pl.BlockSpec((NW, BLK), lambda i: (0, i)),
            pl.BlockSpec((1, BLK), lambda i: (0, i)),
            pl.BlockSpec((ACT, BLK), lambda i: (0, i)),
            full((ACT, 1)), full((ACT, 1)), full((ACT, 1)),
            full((B, FEAT)), full((FEAT, 64)), full((1, 64)),
            full((64, 64)), full((1, 64)),
        ],
        out_specs=[
            pl.BlockSpec((ACT, 1), lambda i: (0, 0)),
            pl.BlockSpec((B, 64), lambda i: (0, 0)),
        ],
        out_shape=(
            jax.ShapeDtypeStruct((ACT, 1), jnp.float32),
            jax.ShapeDtypeStruct((B, 64), jnp.float32),
        ),
    )(num2pst, den2p, den2i, num2it, b2t, gammat, betat,
      features, Wv1, bv12, Wv2, bv22)


@jax.jit
def kernel(features, x, edge_index, W1, a1s, a1d, b1, W2, a2s, a2d, b2,
           gamma, beta, Wv1, bv1, Wv2, bv2):
    pad_e = jnp.full((EPAD - E,), PAD_NODE, jnp.int32)
    src3d = jnp.concatenate([edge_index[0], pad_e]).reshape(NW, NCHK, CH)
    dst3d = jnp.concatenate([edge_index[1], pad_e]).reshape(NW, NCHK, CH)
    xp = jnp.concatenate([x[:, 0], jnp.zeros((NP - N,), jnp.float32)])

    denp, snump = _run_pass_a(src3d, dst3d, xp, W1.reshape(HID), a1s, a1d)

    gt, es2, ed2, den2i, num2it = _run_pass_b(
        xp.reshape(1, NP), denp, snump,
        W1.reshape(HID, 1), a1s.reshape(HID, 1), a1d.reshape(HID, 1),
        b1.reshape(HID, 1), W2.T, a2s.reshape(ACT, 1), a2d.reshape(ACT, 1))

    den2p, num2ps = _run_pass_c(src3d, dst3d,
                                es2.reshape(NP), ed2.reshape(NP), gt.T)

    actorT, critic = _run_pass_d(
        jnp.transpose(num2ps, (0, 2, 1)), den2p, den2i, num2it,
        b2.reshape(ACT, 1), gamma.reshape(ACT, 1), beta.reshape(ACT, 1),
        features, Wv1, bv1.reshape(1, 64), Wv2, bv2.reshape(1, 64))
    return (actorT.T, critic)


# speedup vs baseline: 1.3030x; 1.2604x over previous
import functools

import jax
import jax.numpy as jnp
from jax import lax
from jax.experimental import pallas as pl
from jax.experimental.pallas import tpu as pltpu
from jax.experimental.pallas import tpu_sc as plsc

N = 10000
NP = 10240
E = 640000
HID = 128
ACT = 32
FEAT = 256
B = 64

NC = 2
NS = 16
NW = NC * NS
CH = 128
NCHK = 158
EPW = NCHK * CH
EPAD = NW * EPW
PAD_NODE = N
NODES_PER_TILE = NP // NS

_mesh = plsc.VectorSubcoreMesh(core_axis_name="c", subcore_axis_name="s",
                              num_cores=NC, num_subcores=NS)
_sc_params = pltpu.CompilerParams(needs_layout_passes=False,
                                  use_tc_tiling_on_sc=False)


def _leaky(v):
    return jnp.where(v > 0, v, 0.2 * v)


_GATHER_DNUMS = lax.GatherDimensionNumbers(
    offset_dims=(), collapsed_slice_dims=(0,), start_index_map=(0,))


def _splat_lane(vec, j):
    idx = jnp.full((16, 1), j, jnp.int32)
    return lax.gather(vec, idx, _GATHER_DNUMS, (1,),
                      mode=lax.GatherScatterMode.PROMISE_IN_BOUNDS)


def _allsum16(vec):
    lanes = lax.iota(jnp.int32, 16)
    for sh in (8, 4, 2, 1):
        idx = ((lanes + sh) & 15)[:, None]
        vec = vec + lax.gather(vec, idx, _GATHER_DNUMS, (1,),
                               mode=lax.GatherScatterMode.PROMISE_IN_BOUNDS)
    return vec


def _sc_pass_a(src_hbm, dst_hbm, x_hbm, w1_hbm, a1s_hbm, a1d_hbm,
               den_out, snum_out,
               src_v, dst_v, x_v, par_v, acc_d, acc_s):
    cid = lax.axis_index("c")
    sid = lax.axis_index("s")
    wid = cid * NS + sid

    pltpu.sync_copy(src_hbm.at[wid], src_v)
    pltpu.sync_copy(dst_hbm.at[wid], dst_v)
    pltpu.sync_copy(x_hbm, x_v)
    pltpu.sync_copy(w1_hbm, par_v.at[0])
    pltpu.sync_copy(a1s_hbm, par_v.at[1])
    pltpu.sync_copy(a1d_hbm, par_v.at[2])

    def _dot(row):
        def body(i, t):
            return t + par_v[0, pl.ds(i * 16, 16)] * par_v[row, pl.ds(i * 16, 16)]
        return _allsum16(lax.fori_loop(0, HID // 16, body,
                                       jnp.zeros((16,), jnp.float32)))
    c1s = _dot(1)
    c1d = _dot(2)

    def zbody(i, _):
        z = jnp.zeros((16,), jnp.float32)
        acc_d[pl.ds(i * 16, 16)] = z
        acc_s[pl.ds(i * 16, 16)] = z
        return 0
    lax.fori_loop(0, NP // 16, zbody, 0)

    def ebody(ch, _):
        for g in range(CH // 16):
            s16 = src_v[ch, pl.ds(g * 16, 16)]
            d16 = dst_v[ch, pl.ds(g * 16, 16)]
            xs = plsc.load_gather(x_v, [s16])
            xd = plsc.load_gather(x_v, [d16])
            w = jnp.exp(_leaky(c1s * xs + c1d * xd))
            plsc.addupdate_scatter(acc_d, [d16], w)
            plsc.addupdate_scatter(acc_s, [d16], w * xs)
        return 0
    lax.fori_loop(0, NCHK, ebody, 0)

    pltpu.sync_copy(acc_d, den_out.at[wid])
    pltpu.sync_copy(acc_s, snum_out.at[wid])


def _run_pass_a(src3d, dst3d, xp, w1row, a1s, a1d):
    kfn = pl.kernel(
        _sc_pass_a,
        out_type=(
            jax.ShapeDtypeStruct((NW, NP), jnp.float32),
            jax.ShapeDtypeStruct((NW, NP), jnp.float32),
        ),
        mesh=_mesh,
        compiler_params=_sc_params,
        scratch_types=[
            pltpu.VMEM((NCHK, CH), jnp.int32),
            pltpu.VMEM((NCHK, CH), jnp.int32),
            pltpu.VMEM((NP,), jnp.float32),
            pltpu.VMEM((3, HID), jnp.float32),
            pltpu.VMEM((NP,), jnp.float32),
            pltpu.VMEM((NP,), jnp.float32),
        ],
    )
    return kfn(src3d, dst3d, xp, w1row, a1s, a1d)


BLK = 1024


def _tc_pass_b(x_ref, denp_ref, snump_ref, w1t_ref, a1st_ref, a1dt_ref,
               b1t_ref, w2t_ref, a2st_ref, a2dt_ref,
               gt_ref, es2_ref, ed2_ref, den2i_ref, num2it_ref):
    xb = x_ref[...]
    w1t = w1t_ref[...]
    c1s = jnp.sum(w1t * a1st_ref[...])
    c1d = jnp.sum(w1t * a1dt_ref[...])
    den_e = jnp.sum(denp_ref[...], axis=0, keepdims=True)
    snum_e = jnp.sum(snump_ref[...], axis=0, keepdims=True)
    wself = jnp.exp(_leaky((c1s + c1d) * xb))
    den = den_e + wself
    snum = snum_e + wself * xb
    s1 = snum / (den + 1e-16)
    h1t = jax.nn.relu(w1t * s1 + b1t_ref[...])
    gt = jnp.dot(w2t_ref[...], h1t, preferred_element_type=jnp.float32)
    es2 = jnp.sum(gt * a2st_ref[...], axis=0, keepdims=True)
    ed2 = jnp.sum(gt * a2dt_ref[...], axis=0, keepdims=True)
    w2self = jnp.exp(_leaky(es2 + ed2))
    gt_ref[...] = gt
    es2_ref[...] = es2
    ed2_ref[...] = ed2
    den2i_ref[...] = w2self
    num2it_ref[...] = w2self * gt


def _run_pass_b(xrow, denp, snump, w1t, a1st, a1dt, b1t, W2t, a2st, a2dt):
    nblk = NP // BLK
    full = lambda shape: pl.BlockSpec(shape, lambda i: (0,) * len(shape))
    return pl.pallas_call(
        _tc_pass_b,
        grid=(nblk,),
        in_specs=[
            pl.BlockSpec((1, BLK), lambda i: (0, i)),
            pl.BlockSpec((NW, BLK), lambda i: (0, i)),
            pl.BlockSpec((NW, BLK), lambda i: (0, i)),
            full((HID, 1)), full((HID, 1)), full((HID, 1)),
            full((HID, 1)), full((ACT, HID)), full((ACT, 1)), full((ACT, 1)),
        ],
        out_specs=[
            pl.BlockSpec((ACT, BLK), lambda i: (0, i)),
            pl.BlockSpec((1, BLK), lambda i: (0, i)),
            pl.BlockSpec((1, BLK), lambda i: (0, i)),
            pl.BlockSpec((1, BLK), lambda i: (0, i)),
            pl.BlockSpec((ACT, BLK), lambda i: (0, i)),
        ],
        out_shape=(
            jax.ShapeDtypeStruct((ACT, NP), jnp.float32),
            jax.ShapeDtypeStruct((1, NP), jnp.float32),
            jax.ShapeDtypeStruct((1, NP), jnp.float32),
            jax.ShapeDtypeStruct((1, NP), jnp.float32),
            jax.ShapeDtypeStruct((ACT, NP), jnp.float32),
        ),
    )(xrow, denp, snump, w1t, a1st, a1dt, b1t, W2t, a2st, a2dt)


def _sc_pass_c(src_hbm, dst_hbm, es2_hbm, ed2_hbm, g_hbm,
               den_out, num_out,
               src_v, dst_v, es_v, ed_v, acc_d,
               rows_in0, rows_in1, rows_out0, rows_out1, slab_v, num_sh,
               sg0, sg1, ss0, ss1):
    cid = lax.axis_index("c")
    sid = lax.axis_index("s")
    wid = cid * NS + sid

    pltpu.sync_copy(src_hbm.at[wid], src_v)
    pltpu.sync_copy(dst_hbm.at[wid], dst_v)
    pltpu.sync_copy(es2_hbm, es_v)
    pltpu.sync_copy(ed2_hbm, ed_v)

    def zbody(i, _):
        acc_d[pl.ds(i * 16, 16)] = jnp.zeros((16,), jnp.float32)
        return 0
    lax.fori_loop(0, NP // 16, zbody, 0)

    def zslab(i, _):
        slab_v[i // 2, pl.ds((i % 2) * 16, 16)] = jnp.zeros((16,), jnp.float32)
        return 0
    lax.fori_loop(0, NODES_PER_TILE * 2, zslab, 0)
    pltpu.sync_copy(slab_v, num_sh.at[pl.ds(sid * NODES_PER_TILE,
                                            NODES_PER_TILE)])
    plsc.subcore_barrier()

    rows_in = (rows_in0, rows_in1)
    rows_out = (rows_out0, rows_out1)
    sg = (sg0, sg1)
    ss = (ss0, ss1)

    for b in range(2):
        pltpu.async_copy(g_hbm.at[src_v.at[b]], rows_in[b], sg[b])

    def chunk_pair(it, _):
        for b in range(2):
            ch = it * 2 + b
            pltpu.make_async_copy(g_hbm.at[src_v.at[ch]], rows_in[b],
                                  sg[b]).wait()

            for gg in range(CH // 16):
                s16 = src_v[ch, pl.ds(gg * 16, 16)]
                d16 = dst_v[ch, pl.ds(gg * 16, 16)]
                es = plsc.load_gather(es_v, [s16])
                ed = plsc.load_gather(ed_v, [d16])
                w16 = jnp.exp(_leaky(es + ed))
                plsc.addupdate_scatter(acc_d, [d16], w16)
                for j in range(16):
                    e = gg * 16 + j
                    wsp = _splat_lane(w16, j)
                    rows_out[b][e, pl.ds(0, 16)] = (
                        rows_in[b][e, pl.ds(0, 16)] * wsp)
                    rows_out[b][e, pl.ds(16, 16)] = (
                        rows_in[b][e, pl.ds(16, 16)] * wsp)

            @pl.when(ch >= 2)
            def _():
                pltpu.make_async_copy(rows_out[b], num_sh.at[dst_v.at[ch]],
                                      ss[b]).wait()
            pltpu.async_copy(rows_out[b], num_sh.at[dst_v.at[ch]], ss[b],
                             add=True)

            @pl.when(ch + 2 < NCHK)
            def _():
                pltpu.async_copy(g_hbm.at[src_v.at[ch + 2]], rows_in[b],
                                 sg[b])
        return 0
    lax.fori_loop(0, NCHK // 2, chunk_pair, 0)

    for b in range(2):
        pltpu.make_async_copy(rows_out[b], num_sh.at[dst_v.at[NCHK - 2 + b]],
                              ss[b]).wait()

    pltpu.sync_copy(acc_d, den_out.at[wid])
    plsc.subcore_barrier()

    pltpu.sync_copy(num_sh.at[pl.ds(sid * NODES_PER_TILE, NODES_PER_TILE)],
                    slab_v)
    pltpu.sync_copy(slab_v,
                    num_out.at[cid, pl.ds(sid * NODES_PER_TILE,
                                          NODES_PER_TILE)])


def _run_pass_c(src3d, dst3d, es2f, ed2f, g):
    kfn = pl.kernel(
        _sc_pass_c,
        out_type=(
            jax.ShapeDtypeStruct((NW, NP), jnp.float32),
            jax.ShapeDtypeStruct((NC, NP, ACT), jnp.float32),
        ),
        mesh=_mesh,
        compiler_params=_sc_params,
        scratch_types=[
            pltpu.VMEM((NCHK, CH), jnp.int32),
            pltpu.VMEM((NCHK, CH), jnp.int32),
            pltpu.VMEM((NP,), jnp.float32),
            pltpu.VMEM((NP,), jnp.float32),
            pltpu.VMEM((NP,), jnp.float32),
            pltpu.VMEM((CH, ACT), jnp.float32),
            pltpu.VMEM((CH, ACT), jnp.float32),
            pltpu.VMEM((CH, ACT), jnp.float32),
            pltpu.VMEM((CH, ACT), jnp.float32),
            pltpu.VMEM((NODES_PER_TILE, ACT), jnp.float32),
            pltpu.VMEM_SHARED((NP, ACT), jnp.float32),
            pltpu.SemaphoreType.DMA,
            pltpu.SemaphoreType.DMA,
            pltpu.SemaphoreType.DMA,
            pltpu.SemaphoreType.DMA,
        ],
    )
    return kfn(src3d, dst3d, es2f, ed2f, g)


def _tc_pass_d(numps_ref, denp_ref, den2i_ref, num2i_ref, b2t_ref,
               gammat_ref, betat_ref, feat_ref, wv1_ref, bv1_ref,
               wv2_ref, bv2_ref, actor_ref, critic_ref):
    i = pl.program_id(0)
    num = jnp.sum(numps_ref[...], axis=0) + num2i_ref[...]
    den = (jnp.sum(denp_ref[...], axis=0, keepdims=True)
           + den2i_ref[...])
    out2 = num / (den + 1e-16) + b2t_ref[...]
    h2 = jax.nn.relu(out2)
    mu = jnp.mean(h2, axis=0, keepdims=True)
    var = jnp.mean((h2 - mu) ** 2, axis=0, keepdims=True)
    ln = (h2 - mu) / jnp.sqrt(var + 1e-5) * gammat_ref[...] + betat_ref[...]
    cols = i * BLK + lax.broadcasted_iota(jnp.int32, (1, BLK), 1)
    ln = jnp.where(cols < N, ln, 0.0)
    part = jnp.sum(ln, axis=-1, keepdims=True)

    @pl.when(i == 0)
    def _():
        actor_ref[...] = part
        v = jax.nn.relu(jnp.dot(feat_ref[...], wv1_ref[...],
                                preferred_element_type=jnp.float32)
                        + bv1_ref[...])
        critic_ref[...] = jax.nn.relu(
            jnp.dot(v, wv2_ref[...], preferred_element_type=jnp.float32)
            + bv2_ref[...])

    @pl.when(i > 0)
    def _():
        actor_ref[...] = actor_ref[...] + part


def _run_pass_d(num2pst, den2p, den2i, num2it, b2t, gammat, betat,
                features, Wv1, bv12, Wv2, bv22):
    nblk = NP // BLK
    full = lambda shape: pl.BlockSpec(shape, lambda i: (0,) * len(shape))
    return pl.pallas_call(
        _tc_pass_d,
        grid=(nblk,),
        in_specs=[
            pl.BlockSpec((NC, ACT, BLK), lambda i: (0, 0, i)),
            pl.BlockSpec((NW, BLK), lambda i: (0, i)),
            pl.BlockSpec((1, BLK), lambda i: (0, i)),
            pl.BlockSpec((ACT, BLK), lambda i: (0, i)),
            full((ACT, 1)), full((ACT, 1)), full((ACT, 1)),
            full((B, FEAT)), full((FEAT, 64)), full((1, 64)),
            full((64, 64)), full((1, 64)),
        ],
        out_specs=[
            pl.BlockSpec((ACT, 1), lambda i: (0, 0)),
            pl.BlockSpec((B, 64), lambda i: (0, 0)),
        ],
        out_shape=(
            jax.ShapeDtypeStruct((ACT, 1), jnp.float32),
            jax.ShapeDtypeStruct((B, 64), jnp.float32),
        ),
    )(num2pst, den2p, den2i, num2it, b2t, gammat, betat,
      features, Wv1, bv12, Wv2, bv22)


@jax.jit
def kernel(features, x, edge_index, W1, a1s, a1d, b1, W2, a2s, a2d, b2,
           gamma, beta, Wv1, bv1, Wv2, bv2):
    pad_e = jnp.full((EPAD - E,), PAD_NODE, jnp.int32)
    src3d = jnp.concatenate([edge_index[0], pad_e]).reshape(NW, NCHK, CH)
    dst3d = jnp.concatenate([edge_index[1], pad_e]).reshape(NW, NCHK, CH)
    xp = jnp.concatenate([x[:, 0], jnp.zeros((NP - N,), jnp.float32)])

    denp, snump = _run_pass_a(src3d, dst3d, xp, W1.reshape(HID), a1s, a1d)

    gt, es2, ed2, den2i, num2it = _run_pass_b(
        xp.reshape(1, NP), denp, snump,
        W1.reshape(HID, 1), a1s.reshape(HID, 1), a1d.reshape(HID, 1),
        b1.reshape(HID, 1), W2.T, a2s.reshape(ACT, 1), a2d.reshape(ACT, 1))

    den2p, num2ps = _run_pass_c(src3d, dst3d,
                                es2.reshape(NP), ed2.reshape(NP), gt.T)

    actorT, critic = _run_pass_d(
        jnp.transpose(num2ps, (0, 2, 1)), den2p, den2i, num2it,
        b2.reshape(ACT, 1), gamma.reshape(ACT, 1), beta.reshape(ACT, 1),
        features, Wv1, bv1.reshape(1, 64), Wv2, bv2.reshape(1, 64))
    return (actorT.T, critic)


# speedup vs baseline: 1.3106x; 1.0058x over previous
import functools

import jax
import jax.numpy as jnp
from jax import lax
from jax.experimental import pallas as pl
from jax.experimental.pallas import tpu as pltpu
from jax.experimental.pallas import tpu_sc as plsc

N = 10000
NP = 10240
E = 640000
HID = 128
ACT = 32
FEAT = 256
B = 64

NC = 2
NS = 16
NW = NC * NS
CH = 128
NCHK = 158
EPW = NCHK * CH
EPAD = NW * EPW
PAD_NODE = N
NODES_PER_TILE = NP // NS

_mesh = plsc.VectorSubcoreMesh(core_axis_name="c", subcore_axis_name="s",
                              num_cores=NC, num_subcores=NS)
_sc_params = pltpu.CompilerParams(needs_layout_passes=False,
                                  use_tc_tiling_on_sc=False)


def _leaky(v):
    return jnp.where(v > 0, v, 0.2 * v)


_GATHER_DNUMS = lax.GatherDimensionNumbers(
    offset_dims=(), collapsed_slice_dims=(0,), start_index_map=(0,))


def _splat_lane(vec, j):
    idx = jnp.full((16, 1), j, jnp.int32)
    return lax.gather(vec, idx, _GATHER_DNUMS, (1,),
                      mode=lax.GatherScatterMode.PROMISE_IN_BOUNDS)


def _allsum16(vec):
    lanes = lax.iota(jnp.int32, 16)
    for sh in (8, 4, 2, 1):
        idx = ((lanes + sh) & 15)[:, None]
        vec = vec + lax.gather(vec, idx, _GATHER_DNUMS, (1,),
                               mode=lax.GatherScatterMode.PROMISE_IN_BOUNDS)
    return vec


def _sc_pass_a(src_hbm, dst_hbm, x_hbm, w1_hbm, a1s_hbm, a1d_hbm,
               den_out, snum_out,
               src_v, dst_v, x_v, par_v, acc_d, acc_s):
    cid = lax.axis_index("c")
    sid = lax.axis_index("s")
    wid = cid * NS + sid

    pltpu.sync_copy(src_hbm.at[wid], src_v)
    pltpu.sync_copy(dst_hbm.at[wid], dst_v)
    pltpu.sync_copy(x_hbm, x_v)
    pltpu.sync_copy(w1_hbm, par_v.at[0])
    pltpu.sync_copy(a1s_hbm, par_v.at[1])
    pltpu.sync_copy(a1d_hbm, par_v.at[2])

    def _dot(row):
        def body(i, t):
            return t + par_v[0, pl.ds(i * 16, 16)] * par_v[row, pl.ds(i * 16, 16)]
        return _allsum16(lax.fori_loop(0, HID // 16, body,
                                       jnp.zeros((16,), jnp.float32)))
    c1s = _dot(1)
    c1d = _dot(2)

    def zbody(i, _):
        z = jnp.zeros((16,), jnp.float32)
        acc_d[pl.ds(i * 16, 16)] = z
        acc_s[pl.ds(i * 16, 16)] = z
        return 0
    lax.fori_loop(0, NP // 16, zbody, 0)

    def ebody(ch, _):
        for g in range(CH // 16):
            s16 = src_v[ch, pl.ds(g * 16, 16)]
            d16 = dst_v[ch, pl.ds(g * 16, 16)]
            xs = plsc.load_gather(x_v, [s16])
            xd = plsc.load_gather(x_v, [d16])
            w = jnp.exp(_leaky(c1s * xs + c1d * xd))
            plsc.addupdate_scatter(acc_d, [d16], w)
            plsc.addupdate_scatter(acc_s, [d16], w * xs)
        return 0
    lax.fori_loop(0, NCHK, ebody, 0)

    pltpu.sync_copy(acc_d, den_out.at[wid])
    pltpu.sync_copy(acc_s, snum_out.at[wid])


def _run_pass_a(src3d, dst3d, xp, w1row, a1s, a1d):
    kfn = pl.kernel(
        _sc_pass_a,
        out_type=(
            jax.ShapeDtypeStruct((NW, NP), jnp.float32),
            jax.ShapeDtypeStruct((NW, NP), jnp.float32),
        ),
        mesh=_mesh,
        compiler_params=_sc_params,
        scratch_types=[
            pltpu.VMEM((NCHK, CH), jnp.int32),
            pltpu.VMEM((NCHK, CH), jnp.int32),
            pltpu.VMEM((NP,), jnp.float32),
            pltpu.VMEM((3, HID), jnp.float32),
            pltpu.VMEM((NP,), jnp.float32),
            pltpu.VMEM((NP,), jnp.float32),
        ],
    )
    return kfn(src3d, dst3d, xp, w1row, a1s, a1d)


BLK = 1024


def _tc_pass_b(x_ref, denp_ref, snump_ref, w1t_ref, a1st_ref, a1dt_ref,
               b1t_ref, w2t_ref, a2st_ref, a2dt_ref,
               gt_ref, es2_ref, ed2_ref, den2i_ref, num2it_ref):
    xb = x_ref[...]
    w1t = w1t_ref[...]
    c1s = jnp.sum(w1t * a1st_ref[...])
    c1d = jnp.sum(w1t * a1dt_ref[...])
    den_e = jnp.sum(denp_ref[...], axis=0, keepdims=True)
    snum_e = jnp.sum(snump_ref[...], axis=0, keepdims=True)
    wself = jnp.exp(_leaky((c1s + c1d) * xb))
    den = den_e + wself
    snum = snum_e + wself * xb
    s1 = snum / (den + 1e-16)
    h1t = jax.nn.relu(w1t * s1 + b1t_ref[...])
    gt = jnp.dot(w2t_ref[...], h1t, preferred_element_type=jnp.float32)
    es2 = jnp.sum(gt * a2st_ref[...], axis=0, keepdims=True)
    ed2 = jnp.sum(gt * a2dt_ref[...], axis=0, keepdims=True)
    w2self = jnp.exp(_leaky(es2 + ed2))
    gt_ref[...] = gt
    es2_ref[...] = es2
    ed2_ref[...] = ed2
    den2i_ref[...] = w2self
    num2it_ref[...] = w2self * gt


def _run_pass_b(xrow, denp, snump, w1t, a1st, a1dt, b1t, W2t, a2st, a2dt):
    nblk = NP // BLK
    full = lambda shape: pl.BlockSpec(shape, lambda i: (0,) * len(shape))
    return pl.pallas_call(
        _tc_pass_b,
        grid=(nblk,),
        in_specs=[
            pl.BlockSpec((1, BLK), lambda i: (0, i)),
            pl.BlockSpec((NW, BLK), lambda i: (0, i)),
            pl.BlockSpec((NW, BLK), lambda i: (0, i)),
            full((HID, 1)), full((HID, 1)), full((HID, 1)),
            full((HID, 1)), full((ACT, HID)), full((ACT, 1)), full((ACT, 1)),
        ],
        out_specs=[
            pl.BlockSpec((ACT, BLK), lambda i: (0, i)),
            pl.BlockSpec((1, BLK), lambda i: (0, i)),
            pl.BlockSpec((1, BLK), lambda i: (0, i)),
            pl.BlockSpec((1, BLK), lambda i: (0, i)),
            pl.BlockSpec((ACT, BLK), lambda i: (0, i)),
        ],
        out_shape=(
            jax.ShapeDtypeStruct((ACT, NP), jnp.float32),
            jax.ShapeDtypeStruct((1, NP), jnp.float32),
            jax.ShapeDtypeStruct((1, NP), jnp.float32),
            jax.ShapeDtypeStruct((1, NP), jnp.float32),
            jax.ShapeDtypeStruct((ACT, NP), jnp.float32),
        ),
    )(xrow, denp, snump, w1t, a1st, a1dt, b1t, W2t, a2st, a2dt)


def _sc_pass_c(src_hbm, dst_hbm, es2_hbm, ed2_hbm, g_hbm,
               den_out, num_out,
               src_v, dst_v, es_v, ed_v, acc_d,
               rows_in0, rows_in1, rows_out0, rows_out1, slab_v, num_sh,
               sg0, sg1, ss0, ss1):
    cid = lax.axis_index("c")
    sid = lax.axis_index("s")
    wid = cid * NS + sid

    pltpu.sync_copy(src_hbm.at[wid], src_v)
    pltpu.sync_copy(dst_hbm.at[wid], dst_v)
    pltpu.sync_copy(es2_hbm, es_v)
    pltpu.sync_copy(ed2_hbm, ed_v)

    def zbody(i, _):
        acc_d[pl.ds(i * 16, 16)] = jnp.zeros((16,), jnp.float32)
        return 0
    lax.fori_loop(0, NP // 16, zbody, 0)

    def zslab(i, _):
        z = jnp.zeros((16,), jnp.float32)
        slab_v[i, pl.ds(0, 16)] = z
        slab_v[i, pl.ds(16, 16)] = z
        return 0
    lax.fori_loop(0, NODES_PER_TILE, zslab, 0)
    pltpu.sync_copy(slab_v, num_sh.at[pl.ds(sid * NODES_PER_TILE,
                                            NODES_PER_TILE)])
    plsc.subcore_barrier()

    rows_in = (rows_in0, rows_in1)
    rows_out = (rows_out0, rows_out1)
    sg = (sg0, sg1)
    ss = (ss0, ss1)

    for b in range(2):
        pltpu.async_copy(g_hbm.at[src_v.at[b]], rows_in[b], sg[b])

    def chunk_pair(it, _):
        for b in range(2):
            ch = it * 2 + b
            pltpu.make_async_copy(g_hbm.at[src_v.at[ch]], rows_in[b],
                                  sg[b]).wait()

            for gg in range(CH // 16):
                s16 = src_v[ch, pl.ds(gg * 16, 16)]
                d16 = dst_v[ch, pl.ds(gg * 16, 16)]
                es = plsc.load_gather(es_v, [s16])
                ed = plsc.load_gather(ed_v, [d16])
                w16 = jnp.exp(_leaky(es + ed))
                plsc.addupdate_scatter(acc_d, [d16], w16)
                for j in range(16):
                    e = gg * 16 + j
                    wsp = _splat_lane(w16, j)
                    rows_out[b][e, pl.ds(0, 16)] = (
                        rows_in[b][e, pl.ds(0, 16)] * wsp)
                    rows_out[b][e, pl.ds(16, 16)] = (
                        rows_in[b][e, pl.ds(16, 16)] * wsp)

            @pl.when(ch >= 2)
            def _():
                pltpu.make_async_copy(rows_out[b], num_sh.at[dst_v.at[ch]],
                                      ss[b]).wait()
            pltpu.async_copy(rows_out[b], num_sh.at[dst_v.at[ch]], ss[b],
                             add=True)

            @pl.when(ch + 2 < NCHK)
            def _():
                pltpu.async_copy(g_hbm.at[src_v.at[ch + 2]], rows_in[b],
                                 sg[b])
        return 0
    lax.fori_loop(0, NCHK // 2, chunk_pair, 0)

    for b in range(2):
        pltpu.make_async_copy(rows_out[b], num_sh.at[dst_v.at[NCHK - 2 + b]],
                              ss[b]).wait()

    pltpu.sync_copy(acc_d, den_out.at[wid])
    plsc.subcore_barrier()

    pltpu.sync_copy(num_sh.at[pl.ds(sid * NODES_PER_TILE, NODES_PER_TILE)],
                    slab_v)
    pltpu.sync_copy(slab_v,
                    num_out.at[cid, pl.ds(sid * NODES_PER_TILE,
                                          NODES_PER_TILE)])


def _run_pass_c(src3d, dst3d, es2f, ed2f, g):
    kfn = pl.kernel(
        _sc_pass_c,
        out_type=(
            jax.ShapeDtypeStruct((NW, NP), jnp.float32),
            jax.ShapeDtypeStruct((NC, NP, ACT), jnp.float32),
        ),
        mesh=_mesh,
        compiler_params=_sc_params,
        scratch_types=[
            pltpu.VMEM((NCHK, CH), jnp.int32),
            pltpu.VMEM((NCHK, CH), jnp.int32),
            pltpu.VMEM((NP,), jnp.float32),
            pltpu.VMEM((NP,), jnp.float32),
            pltpu.VMEM((NP,), jnp.float32),
            pltpu.VMEM((CH, ACT), jnp.float32),
            pltpu.VMEM((CH, ACT), jnp.float32),
            pltpu.VMEM((CH, ACT), jnp.float32),
            pltpu.VMEM((CH, ACT), jnp.float32),
            pltpu.VMEM((NODES_PER_TILE, ACT), jnp.float32),
            pltpu.VMEM_SHARED((NP, ACT), jnp.float32),
            pltpu.SemaphoreType.DMA,
            pltpu.SemaphoreType.DMA,
            pltpu.SemaphoreType.DMA,
            pltpu.SemaphoreType.DMA,
        ],
    )
    return kfn(src3d, dst3d, es2f, ed2f, g)


def _tc_pass_d(numps_ref, denp_ref, den2i_ref, num2i_ref, b2t_ref,
               gammat_ref, betat_ref, feat_ref, wv1_ref, bv1_ref,
               wv2_ref, bv2_ref, actor_ref, critic_ref):
    i = pl.program_id(0)
    num = jnp.sum(numps_ref[...], axis=0) + num2i_ref[...]
    den = (jnp.sum(denp_ref[...], axis=0, keepdims=True)
           + den2i_ref[...])
    out2 = num / (den + 1e-16) + b2t_ref[...]
    h2 = jax.nn.relu(out2)
    mu = jnp.mean(h2, axis=0, keepdims=True)
    var = jnp.mean((h2 - mu) ** 2, axis=0, keepdims=True)
    ln = (h2 - mu) / jnp.sqrt(var + 1e-5) * gammat_ref[...] + betat_ref[...]
    cols = i * BLK + lax.broadcasted_iota(jnp.int32, (1, BLK), 1)
    ln = jnp.where(cols < N, ln, 0.0)
    part = jnp.sum(ln, axis=-1, keepdims=True)

    @pl.when(i == 0)
    def _():
        actor_ref[...] = part
        v = jax.nn.relu(jnp.dot(feat_ref[...], wv1_ref[...],
                                preferred_element_type=jnp.float32)
                        + bv1_ref[...])
        critic_ref[...] = jax.nn.relu(
            jnp.dot(v, wv2_ref[...], preferred_element_type=jnp.float32)
            + bv2_ref[...])

    @pl.when(i > 0)
    def _():
        actor_ref[...] = actor_ref[...] + part


def _run_pass_d(num2pst, den2p, den2i, num2it, b2t, gammat, betat,
                features, Wv1, bv12, Wv2, bv22):
    nblk = NP // BLK
    full = lambda shape: pl.BlockSpec(shape, lambda i: (0,) * len(shape))
    return pl.pallas_call(
        _tc_pass_d,
        grid=(nblk,),
        in_specs=[
            pl.BlockSpec((NC, ACT, BLK), lambda i: (0, 0, i)),
            pl.BlockSpec((NW, BLK), lambda i: (0, i)),
            pl.BlockSpec((1, BLK), lambda i: (0, i)),
            pl.BlockSpec((ACT, BLK), lambda i: (0, i)),
            full((ACT, 1)), full((ACT, 1)), full((ACT, 1)),
            full((B, FEAT)), full((FEAT, 64)), full((1, 64)),
            full((64, 64)), full((1, 64)),
        ],
        out_specs=[
            pl.BlockSpec((ACT, 1), lambda i: (0, 0)),
            pl.BlockSpec((B, 64), lambda i: (0, 0)),
        ],
        out_shape=(
            jax.ShapeDtypeStruct((ACT, 1), jnp.float32),
            jax.ShapeDtypeStruct((B, 64), jnp.float32),
        ),
    )(num2pst, den2p, den2i, num2it, b2t, gammat, betat,
      features, Wv1, bv12, Wv2, bv22)


@jax.jit
def kernel(features, x, edge_index, W1, a1s, a1d, b1, W2, a2s, a2d, b2,
           gamma, beta, Wv1, bv1, Wv2, bv2):
    pad_e = jnp.full((EPAD - E,), PAD_NODE, jnp.int32)
    src3d = jnp.concatenate([edge_index[0], pad_e]).reshape(NW, NCHK, CH)
    dst3d = jnp.concatenate([edge_index[1], pad_e]).reshape(NW, NCHK, CH)
    xp = jnp.concatenate([x[:, 0], jnp.zeros((NP - N,), jnp.float32)])

    denp, snump = _run_pass_a(src3d, dst3d, xp, W1.reshape(HID), a1s, a1d)

    gt, es2, ed2, den2i, num2it = _run_pass_b(
        xp.reshape(1, NP), denp, snump,
        W1.reshape(HID, 1), a1s.reshape(HID, 1), a1d.reshape(HID, 1),
        b1.reshape(HID, 1), W2.T, a2s.reshape(ACT, 1), a2d.reshape(ACT, 1))

    den2p, num2ps = _run_pass_c(src3d, dst3d,
                                es2.reshape(NP), ed2.reshape(NP), gt.T)

    actorT, critic = _run_pass_d(
        jnp.transpose(num2ps, (0, 2, 1)), den2p, den2i, num2it,
        b2.reshape(ACT, 1), gamma.reshape(ACT, 1), beta.reshape(ACT, 1),
        features, Wv1, bv1.reshape(1, 64), Wv2, bv2.reshape(1, 64))
    return (actorT.T, critic)
